# baseline (device time: 106599 ns/iter reference)
import jax
import jax.numpy as jnp
from jax import lax
from jax.experimental import pallas as pl
from jax.experimental.pallas import tpu as pltpu

N_DEV = 32
M_PER = 128
SUB = M_PER // 2
K = 4096
N_PER = 64

HOPS = 16


def _hamiltonian_ring():
    cyc = []
    for z in range(4):
        ys = range(4) if z % 2 == 0 else range(3, -1, -1)
        cyc += [(1, y, z) for y in ys]
    for z in range(3, -1, -1):
        ys = range(4) if z % 2 == 1 else range(3, -1, -1)
        cyc += [(0, y, z) for y in ys]

    def lidx(x, y, z):
        return 8 * z + 2 * y + (x if y % 2 == 0 else 1 - x)

    ring = [lidx(*c) for c in cyc]
    assert sorted(ring) == list(range(N_DEV))
    nxt = [0] * N_DEV
    prv = [0] * N_DEV
    for j, m in enumerate(ring):
        nxt[m] = ring[(j + 1) % N_DEV]
        prv[m] = ring[(j - 1) % N_DEV]
    return nxt, prv


def kernel(x, w_mat, scale_x, scale_w):
    nxt_l, prv_l = _hamiltonian_ring()
    nxt_tbl = jnp.asarray(nxt_l, dtype=jnp.int32)
    prv_tbl = jnp.asarray(prv_l, dtype=jnp.int32)
    sx = scale_x.reshape(1, 1)
    sw = scale_w.reshape(1, 1)

    def body(x_ref, w_ref, sx_ref, sw_ref, nxt_ref, prv_ref, out_ref,
             comm_ref, w16_ref, rs_send, rs_recv, ls_send, ls_recv):
        my = lax.axis_index("i")
        left = prv_ref[my]
        right = nxt_ref[my]

        barrier = pltpu.get_barrier_semaphore()
        for nbr in (left, right):
            pl.semaphore_signal(
                barrier, inc=1,
                device_id=(nbr,), device_id_type=pl.DeviceIdType.MESH,
            )
        pl.semaphore_wait(barrier, 2)

        comm_ref[pl.ds(my * M_PER, M_PER), :] = x_ref[...].astype(
            jnp.float8_e4m3fn)

        def send(origin, sub, dev, send_sem, recv_sem):
            sl = pl.ds(origin * M_PER + sub * SUB, SUB)
            rdma = pltpu.make_async_remote_copy(
                src_ref=comm_ref.at[sl, :],
                dst_ref=comm_ref.at[sl, :],
                send_sem=send_sem,
                recv_sem=recv_sem,
                device_id=(dev,),
                device_id_type=pl.DeviceIdType.MESH,
            )
            rdma.start()
            return rdma

        sends = []
        for sub in range(2):
            sends.append(send(my, sub, right, rs_send.at[0, sub],
                              rs_recv.at[0, sub]))
            sends.append(send(my, sub, left, ls_send.at[0, sub],
                              ls_recv.at[0, sub]))

        w16_ref[...] = w_ref[...].astype(jnp.bfloat16)
        s = sx_ref[0, 0] * sw_ref[0, 0]

        def gemm(origin):
            sl = pl.ds(origin * M_PER, M_PER)
            a = comm_ref[sl, :].astype(jnp.bfloat16)
            acc = jnp.dot(a, w16_ref[...], preferred_element_type=jnp.float32)
            out_ref[sl, :] = acc * s

        gemm(my)

        def wait_recv(origin, sub, recv_sem):
            sl = pl.ds(origin * M_PER + sub * SUB, SUB)
            rcv = pltpu.make_async_remote_copy(
                src_ref=comm_ref.at[sl, :],
                dst_ref=comm_ref.at[sl, :],
                send_sem=rs_send.at[0, 0],
                recv_sem=recv_sem,
                device_id=(left,),
                device_id_type=pl.DeviceIdType.MESH,
            )
            rcv.wait_recv()

        o_r = my
        o_l = my
        for h in range(1, HOPS):
            o_r = prv_ref[o_r]
            for sub in range(2):
                wait_recv(o_r, sub, rs_recv.at[h - 1, sub])
                if h < HOPS - 1 or sub == 0:
                    sends.append(send(o_r, sub, right, rs_send.at[h, sub],
                                      rs_recv.at[h, sub]))
            o_l = nxt_ref[o_l]
            for sub in range(2):
                wait_recv(o_l, sub, ls_recv.at[h - 1, sub])
                if h < HOPS - 1 or sub == 1:
                    sends.append(send(o_l, sub, left, ls_send.at[h, sub],
                                      ls_recv.at[h, sub]))
            gemm(o_r)
            gemm(o_l)

        o16 = prv_ref[o_r]
        wait_recv(o16, 0, rs_recv.at[HOPS - 1, 0])
        wait_recv(o16, 1, ls_recv.at[HOPS - 1, 1])
        gemm(o16)

        for rdma in sends:
            rdma.wait_send()

    return pl.pallas_call(
        body,
        out_shape=jax.ShapeDtypeStruct((N_DEV * M_PER, N_PER), jnp.float32),
        in_specs=[
            pl.BlockSpec(memory_space=pltpu.VMEM),
            pl.BlockSpec(memory_space=pltpu.VMEM),
            pl.BlockSpec(memory_space=pltpu.SMEM),
            pl.BlockSpec(memory_space=pltpu.SMEM),
            pl.BlockSpec(memory_space=pltpu.SMEM),
            pl.BlockSpec(memory_space=pltpu.SMEM),
        ],
        out_specs=pl.BlockSpec(memory_space=pltpu.VMEM),
        scratch_shapes=[
            pltpu.VMEM((N_DEV * M_PER, K), jnp.float8_e4m3fn),
            pltpu.VMEM((K, N_PER), jnp.bfloat16),
            pltpu.SemaphoreType.DMA((HOPS, 2)),
            pltpu.SemaphoreType.DMA((HOPS, 2)),
            pltpu.SemaphoreType.DMA((HOPS, 2)),
            pltpu.SemaphoreType.DMA((HOPS, 2)),
        ],
        compiler_params=pltpu.CompilerParams(
            collective_id=0,
            vmem_limit_bytes=100 * 1024 * 1024,
        ),
    )(x, w_mat, sx, sw, nxt_tbl, prv_tbl)


# device time: 106400 ns/iter; 1.0019x vs baseline; 1.0019x over previous
import jax
import jax.numpy as jnp
from jax import lax
from jax.experimental import pallas as pl
from jax.experimental.pallas import tpu as pltpu

N_DEV = 32
M_PER = 128
SUB = M_PER // 2
K = 4096
N_PER = 64

HOPS = 16


def _hamiltonian_ring():
    cyc = []
    for z in range(4):
        ys = range(4) if z % 2 == 0 else range(3, -1, -1)
        cyc += [(1, y, z) for y in ys]
    for z in range(3, -1, -1):
        ys = range(4) if z % 2 == 1 else range(3, -1, -1)
        cyc += [(0, y, z) for y in ys]

    def lidx(x, y, z):
        return 8 * z + 2 * y + (x if y % 2 == 0 else 1 - x)

    ring = [lidx(*c) for c in cyc]
    assert sorted(ring) == list(range(N_DEV))
    nxt = [0] * N_DEV
    prv = [0] * N_DEV
    for j, m in enumerate(ring):
        nxt[m] = ring[(j + 1) % N_DEV]
        prv[m] = ring[(j - 1) % N_DEV]
    return nxt, prv


def kernel(x, w_mat, scale_x, scale_w):
    nxt_l, prv_l = _hamiltonian_ring()
    nxt_tbl = jnp.asarray(nxt_l, dtype=jnp.int32)
    prv_tbl = jnp.asarray(prv_l, dtype=jnp.int32)
    sx = scale_x.reshape(1, 1)
    sw = scale_w.reshape(1, 1)

    def body(x_ref, w_ref, sx_ref, sw_ref, nxt_ref, prv_ref, out_ref,
             comm_ref, w8_ref, rs_send, rs_recv, ls_send, ls_recv):
        my = lax.axis_index("i")
        left = prv_ref[my]
        right = nxt_ref[my]

        barrier = pltpu.get_barrier_semaphore()
        for nbr in (left, right):
            pl.semaphore_signal(
                barrier, inc=1,
                device_id=(nbr,), device_id_type=pl.DeviceIdType.MESH,
            )
        pl.semaphore_wait(barrier, 2)

        def send(origin, sub, dev, send_sem, recv_sem):
            sl = pl.ds(origin * M_PER + sub * SUB, SUB)
            rdma = pltpu.make_async_remote_copy(
                src_ref=comm_ref.at[sl, :],
                dst_ref=comm_ref.at[sl, :],
                send_sem=send_sem,
                recv_sem=recv_sem,
                device_id=(dev,),
                device_id_type=pl.DeviceIdType.MESH,
            )
            rdma.start()
            return rdma

        sends = []
        for sub in range(2):
            sl = pl.ds(sub * SUB, SUB)
            comm_ref[pl.ds(my * M_PER + sub * SUB, SUB), :] = x_ref[
                sl, :].astype(jnp.float8_e4m3fn)
            sends.append(send(my, sub, right, rs_send.at[0, sub],
                              rs_recv.at[0, sub]))
            sends.append(send(my, sub, left, ls_send.at[0, sub],
                              ls_recv.at[0, sub]))

        w8_ref[...] = w_ref[...].astype(jnp.float8_e4m3fn)
        s = sx_ref[0, 0] * sw_ref[0, 0]

        def gemm(origin):
            sl = pl.ds(origin * M_PER, M_PER)
            acc = jnp.dot(comm_ref[sl, :], w8_ref[...],
                          preferred_element_type=jnp.float32)
            out_ref[sl, :] = acc * s

        gemm(my)

        def wait_recv(origin, sub, recv_sem):
            sl = pl.ds(origin * M_PER + sub * SUB, SUB)
            rcv = pltpu.make_async_remote_copy(
                src_ref=comm_ref.at[sl, :],
                dst_ref=comm_ref.at[sl, :],
                send_sem=rs_send.at[0, 0],
                recv_sem=recv_sem,
                device_id=(left,),
                device_id_type=pl.DeviceIdType.MESH,
            )
            rcv.wait_recv()

        o_r = my
        o_l = my
        for h in range(1, HOPS):
            o_r = prv_ref[o_r]
            for sub in range(2):
                wait_recv(o_r, sub, rs_recv.at[h - 1, sub])
                if h < HOPS - 1 or sub == 0:
                    sends.append(send(o_r, sub, right, rs_send.at[h, sub],
                                      rs_recv.at[h, sub]))
            o_l = nxt_ref[o_l]
            for sub in range(2):
                wait_recv(o_l, sub, ls_recv.at[h - 1, sub])
                if h < HOPS - 1 or sub == 1:
                    sends.append(send(o_l, sub, left, ls_send.at[h, sub],
                                      ls_recv.at[h, sub]))
            gemm(o_r)
            gemm(o_l)

        o16 = prv_ref[o_r]
        wait_recv(o16, 0, rs_recv.at[HOPS - 1, 0])
        wait_recv(o16, 1, ls_recv.at[HOPS - 1, 1])
        gemm(o16)

        for rdma in sends:
            rdma.wait_send()

    return pl.pallas_call(
        body,
        out_shape=jax.ShapeDtypeStruct((N_DEV * M_PER, N_PER), jnp.float32),
        in_specs=[
            pl.BlockSpec(memory_space=pltpu.VMEM),
            pl.BlockSpec(memory_space=pltpu.VMEM),
            pl.BlockSpec(memory_space=pltpu.SMEM),
            pl.BlockSpec(memory_space=pltpu.SMEM),
            pl.BlockSpec(memory_space=pltpu.SMEM),
            pl.BlockSpec(memory_space=pltpu.SMEM),
        ],
        out_specs=pl.BlockSpec(memory_space=pltpu.VMEM),
        scratch_shapes=[
            pltpu.VMEM((N_DEV * M_PER, K), jnp.float8_e4m3fn),
            pltpu.VMEM((K, N_PER), jnp.float8_e4m3fn),
            pltpu.SemaphoreType.DMA((HOPS, 2)),
            pltpu.SemaphoreType.DMA((HOPS, 2)),
            pltpu.SemaphoreType.DMA((HOPS, 2)),
            pltpu.SemaphoreType.DMA((HOPS, 2)),
        ],
        compiler_params=pltpu.CompilerParams(
            collective_id=0,
            vmem_limit_bytes=100 * 1024 * 1024,
        ),
    )(x, w_mat, sx, sw, nxt_tbl, prv_tbl)


# device time: 105042 ns/iter; 1.0148x vs baseline; 1.0129x over previous
import jax
import jax.numpy as jnp
from jax import lax
from jax.experimental import pallas as pl
from jax.experimental.pallas import tpu as pltpu

N_DEV = 32
M_PER = 128
SUBS = 4
SUB = M_PER // SUBS
K = 4096
N_PER = 64

HOPS = 16
RS_LAST = (0, 1)
LS_LAST = (2, 3)


def _hamiltonian_ring():
    cyc = []
    for z in range(4):
        ys = range(4) if z % 2 == 0 else range(3, -1, -1)
        cyc += [(1, y, z) for y in ys]
    for z in range(3, -1, -1):
        ys = range(4) if z % 2 == 1 else range(3, -1, -1)
        cyc += [(0, y, z) for y in ys]

    def lidx(x, y, z):
        return 8 * z + 2 * y + (x if y % 2 == 0 else 1 - x)

    ring = [lidx(*c) for c in cyc]
    assert sorted(ring) == list(range(N_DEV))
    nxt = [0] * N_DEV
    prv = [0] * N_DEV
    for j, m in enumerate(ring):
        nxt[m] = ring[(j + 1) % N_DEV]
        prv[m] = ring[(j - 1) % N_DEV]
    return nxt, prv


def kernel(x, w_mat, scale_x, scale_w):
    nxt_l, prv_l = _hamiltonian_ring()
    nxt_tbl = jnp.asarray(nxt_l, dtype=jnp.int32)
    prv_tbl = jnp.asarray(prv_l, dtype=jnp.int32)
    sx = scale_x.reshape(1, 1)
    sw = scale_w.reshape(1, 1)

    def body(x_ref, w_ref, sx_ref, sw_ref, nxt_ref, prv_ref, out_ref,
             comm_ref, w8_ref, rs_send, rs_recv, ls_send, ls_recv):
        my = lax.axis_index("i")
        left = prv_ref[my]
        right = nxt_ref[my]

        barrier = pltpu.get_barrier_semaphore()
        for nbr in (left, right):
            pl.semaphore_signal(
                barrier, inc=1,
                device_id=(nbr,), device_id_type=pl.DeviceIdType.MESH,
            )
        pl.semaphore_wait(barrier, 2)

        def send(origin, sub, dev, send_sem, recv_sem):
            sl = pl.ds(origin * M_PER + sub * SUB, SUB)
            rdma = pltpu.make_async_remote_copy(
                src_ref=comm_ref.at[sl, :],
                dst_ref=comm_ref.at[sl, :],
                send_sem=send_sem,
                recv_sem=recv_sem,
                device_id=(dev,),
                device_id_type=pl.DeviceIdType.MESH,
            )
            rdma.start()
            return rdma

        hop_sends = []
        h0 = []
        for sub in range(SUBS):
            sl = pl.ds(sub * SUB, SUB)
            comm_ref[pl.ds(my * M_PER + sub * SUB, SUB), :] = x_ref[
                sl, :].astype(jnp.float8_e4m3fn)
            h0.append(send(my, sub, right, rs_send.at[0, sub],
                           rs_recv.at[0, sub]))
            h0.append(send(my, sub, left, ls_send.at[0, sub],
                           ls_recv.at[0, sub]))
        hop_sends.append(h0)

        w8_ref[...] = w_ref[...].astype(jnp.float8_e4m3fn)
        s = sx_ref[0, 0] * sw_ref[0, 0]

        def gemm(origin):
            sl = pl.ds(origin * M_PER, M_PER)
            acc = jnp.dot(comm_ref[sl, :], w8_ref[...],
                          preferred_element_type=jnp.float32)
            out_ref[sl, :] = acc * s

        gemm(my)

        def wait_recv(origin, sub, recv_sem):
            sl = pl.ds(origin * M_PER + sub * SUB, SUB)
            rcv = pltpu.make_async_remote_copy(
                src_ref=comm_ref.at[sl, :],
                dst_ref=comm_ref.at[sl, :],
                send_sem=rs_send.at[0, 0],
                recv_sem=recv_sem,
                device_id=(left,),
                device_id_type=pl.DeviceIdType.MESH,
            )
            rcv.wait_recv()

        o_r = my
        o_l = my
        for h in range(1, HOPS):
            if h >= 2:
                for rdma in hop_sends[h - 2]:
                    rdma.wait_send()
            hs = []
            o_r = prv_ref[o_r]
            for sub in range(SUBS):
                wait_recv(o_r, sub, rs_recv.at[h - 1, sub])
                if h < HOPS - 1 or sub in RS_LAST:
                    hs.append(send(o_r, sub, right, rs_send.at[h % 2, sub],
                                   rs_recv.at[h, sub]))
            o_l = nxt_ref[o_l]
            for sub in range(SUBS):
                wait_recv(o_l, sub, ls_recv.at[h - 1, sub])
                if h < HOPS - 1 or sub in LS_LAST:
                    hs.append(send(o_l, sub, left, ls_send.at[h % 2, sub],
                                   ls_recv.at[h, sub]))
            hop_sends.append(hs)
            gemm(o_r)
            gemm(o_l)

        o16 = prv_ref[o_r]
        for sub in RS_LAST:
            wait_recv(o16, sub, rs_recv.at[HOPS - 1, sub])
        for sub in LS_LAST:
            wait_recv(o16, sub, ls_recv.at[HOPS - 1, sub])
        gemm(o16)

        for hs in hop_sends[HOPS - 2:]:
            for rdma in hs:
                rdma.wait_send()

    return pl.pallas_call(
        body,
        out_shape=jax.ShapeDtypeStruct((N_DEV * M_PER, N_PER), jnp.float32),
        in_specs=[
            pl.BlockSpec(memory_space=pltpu.VMEM),
            pl.BlockSpec(memory_space=pltpu.VMEM),
            pl.BlockSpec(memory_space=pltpu.SMEM),
            pl.BlockSpec(memory_space=pltpu.SMEM),
            pl.BlockSpec(memory_space=pltpu.SMEM),
            pl.BlockSpec(memory_space=pltpu.SMEM),
        ],
        out_specs=pl.BlockSpec(memory_space=pltpu.VMEM),
        scratch_shapes=[
            pltpu.VMEM((N_DEV * M_PER, K), jnp.float8_e4m3fn),
            pltpu.VMEM((K, N_PER), jnp.float8_e4m3fn),
            pltpu.SemaphoreType.DMA((2, SUBS)),
            pltpu.SemaphoreType.DMA((HOPS, SUBS)),
            pltpu.SemaphoreType.DMA((2, SUBS)),
            pltpu.SemaphoreType.DMA((HOPS, SUBS)),
        ],
        compiler_params=pltpu.CompilerParams(
            collective_id=0,
            vmem_limit_bytes=100 * 1024 * 1024,
        ),
    )(x, w_mat, sx, sw, nxt_tbl, prv_tbl)
